# baseline (speedup 1.0000x reference)
import functools

import jax
import jax.numpy as jnp
from jax import lax
from jax.experimental import pallas as pl
from jax.experimental.pallas import tpu as pltpu
from jax.experimental.pallas import tpu_sc as plsc

B = 1024
T = 128
OBS = 256
A = 128

_BB = 64
_NW = 32
_ROWS = B // _NW

_vector_mesh = plsc.VectorSubcoreMesh(core_axis_name="c", subcore_axis_name="s")


def _big_body(step_ref, obs_in, mask_in, obs_new, mask_new,
              obs_out, mask_out):
    s = step_ref[0]
    hit3 = jax.lax.broadcasted_iota(jnp.int32, (1, T, 1), 1) == s
    obs_out[...] = jnp.where(hit3, obs_new[...][:, None, :], obs_in[...])
    mask_out[...] = jnp.where(hit3, mask_new[...][:, None, :], mask_in[...])


def _fill_idx(idx_vmem, base, stride, s_vec):
    iot = lax.iota(jnp.int32, 16)
    idx_vmem[pl.ds(0, 16)] = (base + iot) * stride + s_vec
    idx_vmem[pl.ds(16, 16)] = (base + 16 + iot) * stride + s_vec


def _sc_store_body(act_in, rew_in, lp_in, val_in,
                   a_new, r_new, l_new, v_new, step_in,
                   act_out, rew_out, lp_out, val_out,
                   step_v, idx_t, idx_v, upd_v, upd_i,
                   act_st, f32_st, sem):
    wid = lax.axis_index("s") * 2 + lax.axis_index("c")
    base = wid * _ROWS
    pltpu.async_copy(step_in, step_v, sem).wait()
    s_vec = step_v[...]
    rows = pl.ds(base, _ROWS)
    flat_t = pl.ds(base * T, _ROWS * T)
    flat_v = pl.ds(base * (T + 1), _ROWS * (T + 1))

    pltpu.async_copy(act_in.at[flat_t], act_st, sem).wait()
    pltpu.async_copy(act_st, act_out.at[flat_t], sem).wait()
    pltpu.async_copy(rew_in.at[flat_t], f32_st.at[pl.ds(0, _ROWS * T)], sem).wait()
    pltpu.async_copy(f32_st.at[pl.ds(0, _ROWS * T)], rew_out.at[flat_t], sem).wait()
    pltpu.async_copy(lp_in.at[flat_t], f32_st.at[pl.ds(0, _ROWS * T)], sem).wait()
    pltpu.async_copy(f32_st.at[pl.ds(0, _ROWS * T)], lp_out.at[flat_t], sem).wait()
    pltpu.async_copy(val_in.at[flat_v], f32_st, sem).wait()
    pltpu.async_copy(f32_st, val_out.at[flat_v], sem).wait()

    _fill_idx(idx_t, base, T, s_vec)
    _fill_idx(idx_v, base, T + 1, s_vec)

    pltpu.async_copy(a_new.at[rows], upd_i, sem).wait()
    pltpu.async_copy(upd_i, act_out.at[idx_t], sem).wait()
    pltpu.async_copy(r_new.at[rows], upd_v, sem).wait()
    pltpu.async_copy(upd_v, rew_out.at[idx_t], sem).wait()
    pltpu.async_copy(l_new.at[rows], upd_v, sem).wait()
    pltpu.async_copy(upd_v, lp_out.at[idx_t], sem).wait()
    pltpu.async_copy(v_new.at[rows], upd_v, sem).wait()
    pltpu.async_copy(upd_v, val_out.at[idx_v], sem).wait()


_sc_store = functools.partial(
    pl.kernel,
    out_type=(
        jax.ShapeDtypeStruct((B * T,), jnp.int32),
        jax.ShapeDtypeStruct((B * T,), jnp.float32),
        jax.ShapeDtypeStruct((B * T,), jnp.float32),
        jax.ShapeDtypeStruct((B * (T + 1),), jnp.float32),
    ),
    mesh=_vector_mesh,
    scratch_types=[
        pltpu.VMEM((16,), jnp.int32),
        pltpu.VMEM((_ROWS,), jnp.int32),
        pltpu.VMEM((_ROWS,), jnp.int32),
        pltpu.VMEM((_ROWS,), jnp.float32),
        pltpu.VMEM((_ROWS,), jnp.int32),
        pltpu.VMEM((_ROWS * T,), jnp.int32),
        pltpu.VMEM((_ROWS * (T + 1),), jnp.float32),
        pltpu.SemaphoreType.DMA,
    ],
)(_sc_store_body)


def kernel(state_obs, state_action_mask, state_actions, state_rewards,
           state_log_prob, state_values, state_advantages, state_targets,
           step, obs, action_mask, action, reward, log_prob, value):
    step_arr = jnp.asarray(step, jnp.int32).reshape((1,))

    new_act, new_rew, new_lp, new_val = _sc_store(
        state_actions.reshape(B * T),
        state_rewards.reshape(B * T),
        state_log_prob.reshape(B * T),
        state_values.reshape(B * (T + 1)),
        action, reward, log_prob, value,
        jnp.full((16,), jnp.asarray(step, jnp.int32), jnp.int32))

    slide3 = lambda t_, a_: pl.BlockSpec((_BB, t_, a_), lambda i: (i, 0, 0))
    slide2 = lambda t_: pl.BlockSpec((_BB, t_), lambda i: (i, 0))

    new_obs, new_mask = pl.pallas_call(
        _big_body,
        grid=(B // _BB,),
        in_specs=[
            pl.BlockSpec(memory_space=pltpu.SMEM),
            slide3(T, OBS), slide3(T, A),
            slide2(OBS), slide2(A),
        ],
        out_specs=[slide3(T, OBS), slide3(T, A)],
        out_shape=(
            jax.ShapeDtypeStruct((B, T, OBS), jnp.float32),
            jax.ShapeDtypeStruct((B, T, A), jnp.int8),
        ),
    )(step_arr, state_obs, state_action_mask.astype(jnp.int8),
      obs, action_mask.astype(jnp.int8))

    return (new_obs, new_mask.astype(jnp.bool_),
            new_act.reshape(B, T), new_rew.reshape(B, T),
            new_lp.reshape(B, T), new_val.reshape(B, T + 1),
            state_advantages, state_targets)


# speedup vs baseline: 1.0935x; 1.0935x over previous
import jax
import jax.numpy as jnp
from jax.experimental import pallas as pl
from jax.experimental.pallas import tpu as pltpu

B = 1024
T = 128
OBS = 256
A = 128

_BB = 64


def _body(step_ref,
          obs_in, mask_in, act_in, rew_in, lp_in, val_in,
          obs_new, mask_new, a_new, r_new, l_new, v_new,
          obs_out, mask_out, act_out, rew_out, lp_out, val_out):
    s = step_ref[0]
    hit3 = jax.lax.broadcasted_iota(jnp.int32, (1, T, 1), 1) == s
    obs_out[...] = jnp.where(hit3, obs_new[...][:, None, :], obs_in[...])
    mask_out[...] = jnp.where(hit3, mask_new[...][:, None, :], mask_in[...])
    hit2 = jax.lax.broadcasted_iota(jnp.int32, (1, T), 1) == s
    act_out[...] = jnp.where(hit2, a_new[...], act_in[...])
    rew_out[...] = jnp.where(hit2, r_new[...], rew_in[...])
    lp_out[...] = jnp.where(hit2, l_new[...], lp_in[...])
    hit2v = jax.lax.broadcasted_iota(jnp.int32, (1, T + 1), 1) == s
    val_out[...] = jnp.where(hit2v, v_new[...], val_in[...])


def kernel(state_obs, state_action_mask, state_actions, state_rewards,
           state_log_prob, state_values, state_advantages, state_targets,
           step, obs, action_mask, action, reward, log_prob, value):
    step_arr = jnp.asarray(step, jnp.int32).reshape((1,))

    slide3 = lambda t_, a_: pl.BlockSpec((_BB, t_, a_), lambda i: (i, 0, 0))
    slide2 = lambda t_: pl.BlockSpec((_BB, t_), lambda i: (i, 0))
    full2 = lambda t_: pl.BlockSpec((B, t_), lambda i: (0, 0))

    outs = pl.pallas_call(
        _body,
        grid=(B // _BB,),
        in_specs=[
            pl.BlockSpec(memory_space=pltpu.SMEM),
            slide3(T, OBS), slide3(T, A),
            full2(T), full2(T), full2(T), full2(T + 1),
            slide2(OBS), slide2(A),
            full2(1), full2(1), full2(1), full2(1),
        ],
        out_specs=[
            slide3(T, OBS), slide3(T, A),
            full2(T), full2(T), full2(T), full2(T + 1),
        ],
        out_shape=(
            jax.ShapeDtypeStruct((B, T, OBS), jnp.float32),
            jax.ShapeDtypeStruct((B, T, A), jnp.int8),
            jax.ShapeDtypeStruct((B, T), jnp.int32),
            jax.ShapeDtypeStruct((B, T), jnp.float32),
            jax.ShapeDtypeStruct((B, T), jnp.float32),
            jax.ShapeDtypeStruct((B, T + 1), jnp.float32),
        ),
    )(step_arr,
      state_obs, state_action_mask.astype(jnp.int8),
      state_actions, state_rewards, state_log_prob, state_values,
      obs, action_mask.astype(jnp.int8),
      action.reshape(B, 1), reward.reshape(B, 1),
      log_prob.reshape(B, 1), value.reshape(B, 1))

    new_obs, new_mask, new_act, new_rew, new_lp, new_val = outs
    return (new_obs, new_mask.astype(jnp.bool_), new_act, new_rew, new_lp,
            new_val, state_advantages, state_targets)
